# baseline (speedup 1.0000x reference)
import functools

import jax
import jax.numpy as jnp
from jax import lax
from jax.experimental import pallas as pl
from jax.experimental.pallas import tpu as pltpu
from jax.experimental.pallas import tpu_sc as plsc

_NC = 2
_NS = 16
_NW = _NC * _NS
_L = 16

_CH = 80
_NBUF = 8
_H = _NBUF // 2


def _sq_norms(z):
    n, d = z.shape

    def body(z_ref, o_ref):
        zz = z_ref[...]
        o_ref[...] = jnp.sum(zz * zz, axis=1)

    return pl.pallas_call(
        body, out_shape=jax.ShapeDtypeStruct((n,), jnp.float32)
    )(z)


def _gae_decode(z, n2, src_idx, dst_idx):
    n, d = z.shape
    e = src_idx.shape[0]
    epw = e // _NW
    nchunk = epw // _CH
    groups = _CH // _L

    mesh = plsc.VectorSubcoreMesh(core_axis_name="c", subcore_axis_name="s")

    @functools.partial(
        pl.kernel,
        mesh=mesh,
        compiler_params=pltpu.CompilerParams(needs_layout_passes=False),
        out_type=jax.ShapeDtypeStruct((e,), jnp.float32),
        scratch_types=[
            pltpu.VMEM((epw,), jnp.int32),
            pltpu.VMEM((epw,), jnp.int32),
            pltpu.VMEM((n,), jnp.float32),
            pltpu.VMEM((epw,), jnp.float32),
        ]
        + [pltpu.VMEM((_CH, d), jnp.float32) for _ in range(_NBUF)]
        + [pltpu.SemaphoreType.DMA for _ in range(_NBUF)],
    )
    def decode(z_hbm, n2_hbm, sidx_hbm, didx_hbm, out_hbm,
               sidx_v, didx_v, n2_v, out_v, *bufs_and_sems):
        wid = lax.axis_index("s") * _NC + lax.axis_index("c")
        wbase = wid * epw

        pltpu.sync_copy(sidx_hbm.at[pl.ds(wbase, epw)], sidx_v)
        pltpu.sync_copy(didx_hbm.at[pl.ds(wbase, epw)], didx_v)
        pltpu.sync_copy(n2_hbm, n2_v)

        bufs = bufs_and_sems[:_NBUF]
        sems = bufs_and_sems[_NBUF:]
        lane = lax.iota(jnp.int32, _L)

        _HH = _CH // 2

        def issue_g1(c, b):
            pltpu.async_copy(z_hbm.at[sidx_v.at[pl.ds(c * _CH, _HH)]],
                             bufs[b].at[pl.ds(0, _HH)], sems[b])
            pltpu.async_copy(z_hbm.at[sidx_v.at[pl.ds(c * _CH + _HH, _HH)]],
                             bufs[b].at[pl.ds(_HH, _HH)], sems[b])

        def issue_add(c, b):
            pltpu.async_copy(z_hbm.at[didx_v.at[pl.ds(c * _CH, _HH)]],
                             bufs[b].at[pl.ds(0, _HH)], sems[b], add=True)
            pltpu.async_copy(z_hbm.at[didx_v.at[pl.ds(c * _CH + _HH, _HH)]],
                             bufs[b].at[pl.ds(_HH, _HH)], sems[b], add=True)

        def wait(b):
            pltpu.make_async_copy(
                z_hbm.at[sidx_v.at[pl.ds(0, _CH)]], bufs[b], sems[b]
            ).wait()

        def compute(c, b):
            rows = bufs[b]

            def group_body(g, carry):
                e0 = g * _L
                esl = pl.ds(c * _CH + e0, _L)
                sivec = sidx_v[esl]
                divec = didx_v[esl]
                n2s = plsc.load_gather(n2_v, [sivec])
                n2d = plsc.load_gather(n2_v, [divec])
                res = jnp.zeros((_L,), jnp.float32)
                for j in range(_L):
                    ee = e0 + j
                    prods = []
                    for k in range(d // _L):
                        sl = pl.ds(k * _L, _L)
                        v = rows[ee, sl]
                        prods.append(v * v)
                    while len(prods) > 1:
                        prods = [a + b2 for a, b2 in
                                 zip(prods[::2], prods[1::2])]
                    tot = jnp.sum(prods[0])
                    res = jnp.where(lane == j, tot, res)
                dot = (res - n2s - n2d) * 0.5
                out_v[esl] = 1.0 / (1.0 + jnp.exp(-dot))
                return carry

            lax.fori_loop(0, groups, group_body, 0)

        for t in range(_H):
            issue_g1(t, t)
        for t in range(_H):
            wait(t)
            issue_add(t, t)
        for t in range(_H, _NBUF):
            issue_g1(t, t)

        def step(x, b):
            @pl.when(x + _H < nchunk)
            def _():
                wait((b + _H) % _NBUF)
                issue_add(x + _H, (b + _H) % _NBUF)

            wait(b)
            compute(x, b)

            @pl.when(x + _NBUF < nchunk)
            def _():
                issue_g1(x + _NBUF, b)

        def quad_body(i, carry):
            for t in range(_NBUF):
                step(i * _NBUF + t, t)
            return carry

        lax.fori_loop(0, nchunk // _NBUF, quad_body, 0)
        for t in range(nchunk % _NBUF):
            step((nchunk // _NBUF) * _NBUF + t, t)

        pltpu.sync_copy(out_v, out_hbm.at[pl.ds(wbase, epw)])

    return decode(z, n2, src_idx, dst_idx)


def kernel(z, edge_index):
    zf = z.astype(jnp.float32)
    ei = edge_index.astype(jnp.int32)
    return _gae_decode(zf, _sq_norms(zf), ei[0], ei[1])


# speedup vs baseline: 1.0038x; 1.0038x over previous
import functools

import jax
import jax.numpy as jnp
from jax import lax
from jax.experimental import pallas as pl
from jax.experimental.pallas import tpu as pltpu
from jax.experimental.pallas import tpu_sc as plsc

_NC = 2
_NS = 16
_NW = _NC * _NS
_L = 16

_CH = 80
_NBUF = 8
_H = _NBUF // 2


def _sq_norms(z):
    n, d = z.shape

    def body(z_ref, o_ref):
        zz = z_ref[...]
        o_ref[...] = jnp.sum(zz * zz, axis=1)

    return pl.pallas_call(
        body, out_shape=jax.ShapeDtypeStruct((n,), jnp.float32)
    )(z)


def _gae_decode(z, n2, src_idx, dst_idx):
    n, d = z.shape
    e = src_idx.shape[0]
    epw = e // _NW
    nchunk = epw // _CH
    groups = _CH // _L

    mesh = plsc.VectorSubcoreMesh(core_axis_name="c", subcore_axis_name="s")

    @functools.partial(
        pl.kernel,
        mesh=mesh,
        compiler_params=pltpu.CompilerParams(needs_layout_passes=False),
        out_type=jax.ShapeDtypeStruct((e,), jnp.float32),
        scratch_types=[
            pltpu.VMEM((epw,), jnp.int32),
            pltpu.VMEM((epw,), jnp.int32),
            pltpu.VMEM((n,), jnp.float32),
            pltpu.VMEM((epw,), jnp.float32),
        ]
        + [pltpu.VMEM((_CH, d), jnp.float32) for _ in range(_NBUF)]
        + [pltpu.SemaphoreType.DMA for _ in range(_NBUF)],
    )
    def decode(z_hbm, n2_hbm, sidx_hbm, didx_hbm, out_hbm,
               sidx_v, didx_v, n2_v, out_v, *bufs_and_sems):
        wid = lax.axis_index("s") * _NC + lax.axis_index("c")
        wbase = wid * epw

        pltpu.sync_copy(sidx_hbm.at[pl.ds(wbase, epw)], sidx_v)
        pltpu.sync_copy(didx_hbm.at[pl.ds(wbase, epw)], didx_v)
        pltpu.sync_copy(n2_hbm, n2_v)

        bufs = bufs_and_sems[:_NBUF]
        sems = bufs_and_sems[_NBUF:]
        lane = lax.iota(jnp.int32, _L)

        def issue_g1(c, b):
            pltpu.async_copy(z_hbm.at[sidx_v.at[pl.ds(c * _CH, _CH)]],
                             bufs[b], sems[b])

        def issue_add(c, b):
            pltpu.async_copy(z_hbm.at[didx_v.at[pl.ds(c * _CH, _CH)]],
                             bufs[b], sems[b], add=True)

        def wait(b):
            pltpu.make_async_copy(
                z_hbm.at[sidx_v.at[pl.ds(0, _CH)]], bufs[b], sems[b]
            ).wait()

        def compute(c, b):
            rows = bufs[b]

            def group_body(g, carry):
                e0 = g * _L
                esl = pl.ds(c * _CH + e0, _L)
                sivec = sidx_v[esl]
                divec = didx_v[esl]
                n2s = plsc.load_gather(n2_v, [sivec])
                n2d = plsc.load_gather(n2_v, [divec])
                res = jnp.zeros((_L,), jnp.float32)
                for j in range(_L):
                    ee = e0 + j
                    prods = []
                    for k in range(d // _L):
                        sl = pl.ds(k * _L, _L)
                        v = rows[ee, sl]
                        prods.append(v * v)
                    while len(prods) > 1:
                        prods = [a + b2 for a, b2 in
                                 zip(prods[::2], prods[1::2])]
                    tot = jnp.sum(prods[0])
                    res = jnp.where(lane == j, tot, res)
                dot = (res - n2s - n2d) * 0.5
                out_v[esl] = 1.0 / (1.0 + jnp.exp(-dot))
                return carry

            lax.fori_loop(0, groups, group_body, 0)

        for t in range(_H):
            issue_g1(t, t)
        for t in range(_H):
            wait(t)
            issue_add(t, t)
        for t in range(_H, _NBUF):
            issue_g1(t, t)

        def step(x, b):
            @pl.when(x + _H < nchunk)
            def _():
                wait((b + _H) % _NBUF)
                issue_add(x + _H, (b + _H) % _NBUF)

            wait(b)
            compute(x, b)

            @pl.when(x + _NBUF < nchunk)
            def _():
                issue_g1(x + _NBUF, b)

        def quad_body(i, carry):
            for t in range(_NBUF):
                step(i * _NBUF + t, t)
            return carry

        lax.fori_loop(0, nchunk // _NBUF, quad_body, 0)
        for t in range(nchunk % _NBUF):
            step((nchunk // _NBUF) * _NBUF + t, t)

        pltpu.sync_copy(out_v, out_hbm.at[pl.ds(wbase, epw)])

    return decode(z, n2, src_idx, dst_idx)


def kernel(z, edge_index):
    zf = z.astype(jnp.float32)
    ei = edge_index.astype(jnp.int32)
    return _gae_decode(zf, _sq_norms(zf), ei[0], ei[1])
